# baseline (speedup 1.0000x reference)
import functools

import jax
import jax.numpy as jnp
from jax.experimental import pallas as pl


def _proj_kernel(x_ref, wt_ref, b_ref, o_ref):
    o_ref[...] = (
        jnp.dot(x_ref[...], wt_ref[...], preferred_element_type=jnp.float32)
        + b_ref[...]
    )


@functools.partial(jax.jit, static_argnames=("bm",))
def _proj(x, wt, b2, bm):
    tot, enc = x.shape
    dec = wt.shape[1]
    return pl.pallas_call(
        _proj_kernel,
        grid=(tot // bm,),
        in_specs=[
            pl.BlockSpec((bm, enc), lambda i: (i, 0)),
            pl.BlockSpec((enc, dec), lambda i: (0, 0)),
            pl.BlockSpec((1, dec), lambda i: (0, 0)),
        ],
        out_specs=pl.BlockSpec((bm, dec), lambda i: (i, 0)),
        out_shape=jax.ShapeDtypeStruct((tot, dec), jnp.float32),
    )(x, wt, b2)


def kernel(x, W, b):
    return _proj(x, W.T, b[None, :], bm=512)


# speedup vs baseline: 1.0001x; 1.0001x over previous
import functools

import jax
import jax.numpy as jnp
from jax.experimental import pallas as pl
from jax.experimental.pallas import tpu as pltpu


def _proj_kernel(x_ref, wt_ref, b_ref, o_ref):
    x_bf = x_ref[...].astype(jnp.bfloat16)
    w_bf = wt_ref[...].astype(jnp.bfloat16)
    o_ref[...] = (
        jnp.dot(x_bf, w_bf, preferred_element_type=jnp.float32) + b_ref[...]
    )


@functools.partial(jax.jit, static_argnames=("bm",))
def _proj(x, wt, b2, bm):
    tot, enc = x.shape
    dec = wt.shape[1]
    return pl.pallas_call(
        _proj_kernel,
        grid=(tot // bm,),
        in_specs=[
            pl.BlockSpec((bm, enc), lambda i: (i, 0)),
            pl.BlockSpec((enc, dec), lambda i: (0, 0)),
            pl.BlockSpec((1, dec), lambda i: (0, 0)),
        ],
        out_specs=pl.BlockSpec((bm, dec), lambda i: (i, 0)),
        out_shape=jax.ShapeDtypeStruct((tot, dec), jnp.float32),
        compiler_params=pltpu.CompilerParams(
            dimension_semantics=("parallel",),
        ),
    )(x, wt, b2)


def kernel(x, W, b):
    return _proj(x, W.T, b[None, :], bm=512)


# speedup vs baseline: 1.0764x; 1.0763x over previous
import functools

import jax
import jax.numpy as jnp
from jax.experimental import pallas as pl
from jax.experimental.pallas import tpu as pltpu


def _proj_kernel(x_ref, w_ref, b_ref, o_ref):
    x_bf = x_ref[...].astype(jnp.bfloat16)
    w_bf = w_ref[...].astype(jnp.bfloat16)
    acc = jax.lax.dot_general(
        x_bf, w_bf,
        dimension_numbers=(((1,), (1,)), ((), ())),
        preferred_element_type=jnp.float32,
    )
    o_ref[...] = acc + b_ref[...]


@functools.partial(jax.jit, static_argnames=("bm",))
def _proj(x, w, b2, bm):
    tot, enc = x.shape
    dec = w.shape[0]
    return pl.pallas_call(
        _proj_kernel,
        grid=(tot // bm,),
        in_specs=[
            pl.BlockSpec((bm, enc), lambda i: (i, 0)),
            pl.BlockSpec((dec, enc), lambda i: (0, 0)),
            pl.BlockSpec((1, dec), lambda i: (0, 0)),
        ],
        out_specs=pl.BlockSpec((bm, dec), lambda i: (i, 0)),
        out_shape=jax.ShapeDtypeStruct((tot, dec), jnp.float32),
        compiler_params=pltpu.CompilerParams(
            dimension_semantics=("parallel",),
        ),
    )(x, w, b2)


def kernel(x, W, b):
    return _proj(x, W, b[None, :], bm=512)


# speedup vs baseline: 1.0803x; 1.0035x over previous
import functools

import jax
import jax.numpy as jnp
from jax.experimental import pallas as pl
from jax.experimental.pallas import tpu as pltpu


def _proj_kernel(x_ref, w_ref, b_ref, o_ref, wbf_ref):
    @pl.when(pl.program_id(0) == 0)
    def _cast_w_once():
        wbf_ref[...] = w_ref[...].astype(jnp.bfloat16)

    x_bf = x_ref[...].astype(jnp.bfloat16)
    acc = jax.lax.dot_general(
        x_bf, wbf_ref[...],
        dimension_numbers=(((1,), (1,)), ((), ())),
        preferred_element_type=jnp.float32,
    )
    o_ref[...] = acc + b_ref[...]


@functools.partial(jax.jit, static_argnames=("bm",))
def _proj(x, w, b2, bm):
    tot, enc = x.shape
    dec = w.shape[0]
    return pl.pallas_call(
        _proj_kernel,
        grid=(tot // bm,),
        in_specs=[
            pl.BlockSpec((bm, enc), lambda i: (i, 0)),
            pl.BlockSpec((dec, enc), lambda i: (0, 0)),
            pl.BlockSpec((1, dec), lambda i: (0, 0)),
        ],
        out_specs=pl.BlockSpec((bm, dec), lambda i: (i, 0)),
        out_shape=jax.ShapeDtypeStruct((tot, dec), jnp.float32),
        scratch_shapes=[pltpu.VMEM((dec, enc), jnp.bfloat16)],
        compiler_params=pltpu.CompilerParams(
            dimension_semantics=("arbitrary",),
        ),
    )(x, w, b2)


def kernel(x, W, b):
    return _proj(x, W, b[None, :], bm=512)


# speedup vs baseline: 1.1230x; 1.0396x over previous
import functools

import jax
import jax.numpy as jnp
from jax.experimental import pallas as pl
from jax.experimental.pallas import tpu as pltpu


def _proj_kernel(x_ref, w_ref, b_ref, o_ref):
    x_bf = x_ref[...].astype(jnp.bfloat16)
    w_bf = w_ref[...].astype(jnp.bfloat16)
    acc = jax.lax.dot_general(
        x_bf, w_bf,
        dimension_numbers=(((1,), (1,)), ((), ())),
        preferred_element_type=jnp.float32,
    )
    o_ref[...] = acc + b_ref[...]


@functools.partial(jax.jit, static_argnames=("bm",))
def _proj(x, w, b2, bm):
    tot, enc = x.shape
    dec = w.shape[0]
    return pl.pallas_call(
        _proj_kernel,
        grid=(tot // bm,),
        in_specs=[
            pl.BlockSpec((bm, enc), lambda i: (i, 0)),
            pl.BlockSpec((dec, enc), lambda i: (0, 0)),
            pl.BlockSpec((1, dec), lambda i: (0, 0)),
        ],
        out_specs=pl.BlockSpec((bm, dec), lambda i: (i, 0)),
        out_shape=jax.ShapeDtypeStruct((tot, dec), jnp.float32),
        compiler_params=pltpu.CompilerParams(
            dimension_semantics=("arbitrary",),
        ),
    )(x, w, b2)


def kernel(x, W, b):
    return _proj(x, W, b[None, :], bm=1024)
